# baseline (speedup 1.0000x reference)
import functools

import jax
import jax.numpy as jnp
from jax import lax
from jax.experimental import pallas as pl
from jax.experimental.pallas import tpu as pltpu
from jax.experimental.pallas import tpu_sc as plsc

_TM = 512


def _dispatch_gather(x, tok):
    n, d = x.shape
    nk = tok.shape[0]
    info = plsc.get_sparse_core_info()
    nw = info.num_cores * info.num_subcores
    rows_w = nk // nw
    mesh = plsc.VectorSubcoreMesh(core_axis_name="c", subcore_axis_name="s")

    @functools.partial(
        pl.kernel, mesh=mesh,
        out_type=jax.ShapeDtypeStruct((nk, d), jnp.float32),
        scratch_types=[
            pltpu.VMEM((rows_w,), jnp.int32),
            pltpu.VMEM((rows_w, d), jnp.float32),
            pltpu.SemaphoreType.DMA,
        ])
    def k(x_hbm, tok_hbm, out_hbm, idx_v, rows_v, sem):
        wid = lax.axis_index("s") * info.num_cores + lax.axis_index("c")
        base = wid * rows_w
        pltpu.sync_copy(tok_hbm.at[pl.ds(base, rows_w)], idx_v)
        pltpu.async_copy(x_hbm.at[idx_v], rows_v, sem).wait()
        pltpu.sync_copy(rows_v, out_hbm.at[pl.ds(base, rows_w)])

    return k(x, tok)


def _combine(routed_out, inv):
    nk, d = routed_out.shape
    n = inv.shape[0] // 2
    info = plsc.get_sparse_core_info()
    nw = info.num_cores * info.num_subcores
    tok_w = n // nw
    tok_c = tok_w // 2
    mesh = plsc.VectorSubcoreMesh(core_axis_name="c", subcore_axis_name="s")

    @functools.partial(
        pl.kernel, mesh=mesh,
        out_type=jax.ShapeDtypeStruct((n, d), jnp.float32),
        scratch_types=[
            pltpu.VMEM((2 * tok_w,), jnp.int32),
            pltpu.VMEM((2 * tok_c, d), jnp.float32),
            pltpu.VMEM((2 * tok_c, d), jnp.float32),
            pltpu.VMEM((tok_c, d), jnp.float32),
            pltpu.SemaphoreType.DMA,
            pltpu.SemaphoreType.DMA,
            pltpu.SemaphoreType.DMA,
        ])
    def k(ro_hbm, inv_hbm, out_hbm, idx_v, rows0_v, rows1_v, out_v,
          gsem, osem0, osem1):
        wid = lax.axis_index("s") * info.num_cores + lax.axis_index("c")
        tbase = wid * tok_w
        pltpu.sync_copy(inv_hbm.at[pl.ds(2 * tbase, 2 * tok_w)], idx_v)
        g0 = pltpu.async_copy(ro_hbm.at[idx_v.at[pl.ds(0, 2 * tok_c)]],
                              rows0_v, gsem)
        g1 = pltpu.async_copy(ro_hbm.at[idx_v.at[pl.ds(2 * tok_c, 2 * tok_c)]],
                              rows1_v, gsem)
        g0.wait()

        @pl.loop(0, tok_c)
        def _(t):
            for j in range(0, d, 16):
                sl = pl.ds(j, 16)
                out_v[t, sl] = rows0_v[2 * t, sl] + rows0_v[2 * t + 1, sl]

        c0 = pltpu.async_copy(out_v, out_hbm.at[pl.ds(tbase, tok_c)], osem0)
        g1.wait()

        @pl.loop(0, tok_c)
        def _(t):
            for j in range(0, d, 16):
                sl = pl.ds(j, 16)
                rows1_v[2 * t, sl] = (rows1_v[2 * t, sl]
                                      + rows1_v[2 * t + 1, sl])

        c0.wait()

        @pl.loop(0, tok_c)
        def _(t):
            for j in range(0, d, 16):
                sl = pl.ds(j, 16)
                out_v[t, sl] = rows1_v[2 * t, sl]

        pltpu.async_copy(out_v, out_hbm.at[pl.ds(tbase + tok_c, tok_c)],
                         osem1).wait()

    return k(routed_out, inv)


def _ffn_body(gid_ref, mt_ref, off_ref, x_ref, s_ref, w1_ref, w3_ref,
              w2_ref, out_ref):
    t = pl.program_id(0)
    e = gid_ref[t]
    m = mt_ref[t]
    start = off_ref[e]
    end = off_ref[e + 1]
    rows = m * _TM + lax.broadcasted_iota(jnp.int32, (_TM, 1), 0)
    mask = (rows >= start) & (rows < end)
    xt = jnp.where(mask, x_ref[...], 0.0).astype(jnp.bfloat16)
    dn = (((1,), (1,)), ((), ()))
    h1 = lax.dot_general(xt, w1_ref[0].astype(jnp.bfloat16), dn,
                         preferred_element_type=jnp.float32)
    h3 = lax.dot_general(xt, w3_ref[0].astype(jnp.bfloat16), dn,
                         preferred_element_type=jnp.float32)
    h = h1 * (1.0 / (1.0 + jnp.exp(-h1))) * h3
    o = lax.dot_general(h.astype(jnp.bfloat16), w2_ref[0].astype(jnp.bfloat16),
                        dn, preferred_element_type=jnp.float32)
    o = o * s_ref[...]

    tm1 = jnp.maximum(t - 1, 0)
    first = jnp.logical_or(t == 0, mt_ref[t] != mt_ref[tm1])

    @pl.when(first)
    def _():
        out_ref[...] = o

    @pl.when(jnp.logical_not(first))
    def _():
        out_ref[...] += o


def _grouped_ffn(routed_in, scores_sorted, w1, w2, w3, gid, mt, off_ext,
                 n_visits):
    nk, d = routed_in.shape
    e, h, _ = w1.shape

    grid_spec = pltpu.PrefetchScalarGridSpec(
        num_scalar_prefetch=3,
        grid=(n_visits,),
        in_specs=[
            pl.BlockSpec((_TM, d), lambda t, g, m, o: (m[t], 0)),
            pl.BlockSpec((_TM, 1), lambda t, g, m, o: (m[t], 0)),
            pl.BlockSpec((1, h, d),
                         lambda t, g, m, o: (jnp.minimum(g[t], e - 1), 0, 0)),
            pl.BlockSpec((1, h, d),
                         lambda t, g, m, o: (jnp.minimum(g[t], e - 1), 0, 0)),
            pl.BlockSpec((1, d, h),
                         lambda t, g, m, o: (jnp.minimum(g[t], e - 1), 0, 0)),
        ],
        out_specs=pl.BlockSpec((_TM, d), lambda t, g, m, o: (m[t], 0)),
    )
    return pl.pallas_call(
        _ffn_body,
        grid_spec=grid_spec,
        out_shape=jax.ShapeDtypeStruct((nk, d), jnp.float32),
        compiler_params=pltpu.CompilerParams(
            dimension_semantics=("arbitrary",)),
    )(gid, mt, off_ext, routed_in, scores_sorted, w1, w3, w2)


def kernel(x, top_scores, selected_experts_indices, w1, w2, w3):
    n, d = x.shape
    k = selected_experts_indices.shape[1]
    e = w1.shape[0]
    nk = n * k
    m_tiles = nk // _TM
    n_visits = m_tiles + e

    flat_e = selected_experts_indices.reshape(-1).astype(jnp.int32)
    sort_idx = jnp.argsort(flat_e, stable=True).astype(jnp.int32)
    tok = sort_idx // k
    inv = jnp.zeros((nk,), jnp.int32).at[sort_idx].set(
        jnp.arange(nk, dtype=jnp.int32))
    counts = jnp.bincount(flat_e, length=e).astype(jnp.int32)
    offsets = jnp.concatenate(
        [jnp.zeros((1,), jnp.int32), jnp.cumsum(counts).astype(jnp.int32)])
    off_ext = jnp.concatenate([offsets, jnp.full((1,), nk, jnp.int32)])

    ft = offsets[:e] // _TM
    lt = jnp.maximum(offsets[1:] - 1, 0) // _TM
    ntiles = jnp.where(counts > 0, lt - ft + 1, 0)
    vb = jnp.concatenate(
        [jnp.zeros((1,), jnp.int32), jnp.cumsum(ntiles).astype(jnp.int32)[:-1]])
    total = vb[-1] + ntiles[-1]
    marks = jnp.zeros((n_visits + 1,), jnp.int32).at[vb].add(1)
    gid = jnp.cumsum(marks)[:n_visits] - 1
    p = jnp.arange(n_visits, dtype=jnp.int32)
    valid = p < total
    gid_c = jnp.clip(gid, 0, e - 1)
    mt = ft[gid_c] + p - vb[gid_c]
    gid = jnp.where(valid, gid, e).astype(jnp.int32)
    mt = jnp.where(valid, jnp.clip(mt, 0, m_tiles - 1),
                   m_tiles - 1).astype(jnp.int32)

    routed_in = _dispatch_gather(x, tok)

    scores_sorted = top_scores.reshape(-1)[sort_idx][:, None]
    routed_out = _grouped_ffn(routed_in, scores_sorted, w1, w2, w3, gid, mt,
                              off_ext, n_visits)

    return _combine(routed_out, inv)


# speedup vs baseline: 1.1729x; 1.1729x over previous
import functools

import jax
import jax.numpy as jnp
from jax import lax
from jax.experimental import pallas as pl
from jax.experimental.pallas import tpu as pltpu
from jax.experimental.pallas import tpu_sc as plsc

_TM = 256


def _dispatch_gather(x, tok):
    n, d = x.shape
    nk = tok.shape[0]
    info = plsc.get_sparse_core_info()
    nw = info.num_cores * info.num_subcores
    rows_w = nk // nw
    mesh = plsc.VectorSubcoreMesh(core_axis_name="c", subcore_axis_name="s")

    @functools.partial(
        pl.kernel, mesh=mesh,
        out_type=jax.ShapeDtypeStruct((nk, d), jnp.float32),
        scratch_types=[
            pltpu.VMEM((rows_w,), jnp.int32),
            pltpu.VMEM((rows_w, d), jnp.float32),
            pltpu.SemaphoreType.DMA,
        ])
    def k(x_hbm, tok_hbm, out_hbm, idx_v, rows_v, sem):
        wid = lax.axis_index("s") * info.num_cores + lax.axis_index("c")
        base = wid * rows_w
        pltpu.sync_copy(tok_hbm.at[pl.ds(base, rows_w)], idx_v)
        pltpu.async_copy(x_hbm.at[idx_v], rows_v, sem).wait()
        pltpu.sync_copy(rows_v, out_hbm.at[pl.ds(base, rows_w)])

    return k(x, tok)


def _combine(routed_out, inv):
    nk, d = routed_out.shape
    n = inv.shape[0] // 2
    info = plsc.get_sparse_core_info()
    nw = info.num_cores * info.num_subcores
    tok_w = n // nw
    tok_c = tok_w // 2
    mesh = plsc.VectorSubcoreMesh(core_axis_name="c", subcore_axis_name="s")

    @functools.partial(
        pl.kernel, mesh=mesh,
        out_type=jax.ShapeDtypeStruct((n, d), jnp.float32),
        scratch_types=[
            pltpu.VMEM((2 * tok_w,), jnp.int32),
            pltpu.VMEM((2 * tok_c, d), jnp.float32),
            pltpu.VMEM((2 * tok_c, d), jnp.float32),
            pltpu.VMEM((tok_c, d), jnp.float32),
            pltpu.SemaphoreType.DMA,
            pltpu.SemaphoreType.DMA,
            pltpu.SemaphoreType.DMA,
        ])
    def k(ro_hbm, inv_hbm, out_hbm, idx_v, rows0_v, rows1_v, out_v,
          gsem, osem0, osem1):
        wid = lax.axis_index("s") * info.num_cores + lax.axis_index("c")
        tbase = wid * tok_w
        pltpu.sync_copy(inv_hbm.at[pl.ds(2 * tbase, 2 * tok_w)], idx_v)
        g0 = pltpu.async_copy(ro_hbm.at[idx_v.at[pl.ds(0, 2 * tok_c)]],
                              rows0_v, gsem)
        g1 = pltpu.async_copy(ro_hbm.at[idx_v.at[pl.ds(2 * tok_c, 2 * tok_c)]],
                              rows1_v, gsem)
        g0.wait()

        @pl.loop(0, tok_c)
        def _(t):
            for j in range(0, d, 16):
                sl = pl.ds(j, 16)
                out_v[t, sl] = rows0_v[2 * t, sl] + rows0_v[2 * t + 1, sl]

        c0 = pltpu.async_copy(out_v, out_hbm.at[pl.ds(tbase, tok_c)], osem0)
        g1.wait()

        @pl.loop(0, tok_c)
        def _(t):
            for j in range(0, d, 16):
                sl = pl.ds(j, 16)
                rows1_v[2 * t, sl] = (rows1_v[2 * t, sl]
                                      + rows1_v[2 * t + 1, sl])

        c0.wait()

        @pl.loop(0, tok_c)
        def _(t):
            for j in range(0, d, 16):
                sl = pl.ds(j, 16)
                out_v[t, sl] = rows1_v[2 * t, sl]

        pltpu.async_copy(out_v, out_hbm.at[pl.ds(tbase + tok_c, tok_c)],
                         osem1).wait()

    return k(routed_out, inv)


_NBUF = 3


def _ffn_body(gid_ref, mt_ref, off_ref, buf_ref, flag_ref, x_ref, s_ref,
              w1_hbm, w3_hbm, w2_hbm, out_ref, w1_v, w3_v, w2_v, sems):
    t = pl.program_id(0)
    nv = pl.num_programs(0)
    ne = w1_hbm.shape[0]

    def issue(v):
        slot = buf_ref[v]
        ev = jnp.minimum(gid_ref[v], ne - 1)
        pltpu.make_async_copy(w1_hbm.at[ev], w1_v.at[slot],
                              sems.at[slot, 0]).start()
        pltpu.make_async_copy(w3_hbm.at[ev], w3_v.at[slot],
                              sems.at[slot, 1]).start()
        pltpu.make_async_copy(w2_hbm.at[ev], w2_v.at[slot],
                              sems.at[slot, 2]).start()

    @pl.when(t == 0)
    def _():
        issue(0)

        @pl.when(jnp.logical_and(1 < nv, flag_ref[1] == 1))
        def _():
            issue(1)

    @pl.when(jnp.logical_and(t + 2 < nv, flag_ref[t + 2] == 1))
    def _():
        issue(t + 2)

    @pl.when(flag_ref[t] == 1)
    def _():
        slot = buf_ref[t]
        ev = jnp.minimum(gid_ref[t], ne - 1)
        pltpu.make_async_copy(w1_hbm.at[ev], w1_v.at[slot],
                              sems.at[slot, 0]).wait()
        pltpu.make_async_copy(w3_hbm.at[ev], w3_v.at[slot],
                              sems.at[slot, 1]).wait()
        pltpu.make_async_copy(w2_hbm.at[ev], w2_v.at[slot],
                              sems.at[slot, 2]).wait()

    e = gid_ref[t]
    m = mt_ref[t]
    slot = buf_ref[t]
    start = off_ref[e]
    end = off_ref[e + 1]
    rows = m * _TM + lax.broadcasted_iota(jnp.int32, (_TM, 1), 0)
    mask = (rows >= start) & (rows < end)
    xt = jnp.where(mask, x_ref[...], 0.0).astype(jnp.bfloat16)
    dn = (((1,), (1,)), ((), ()))
    h1 = lax.dot_general(xt, w1_v[slot].astype(jnp.bfloat16), dn,
                         preferred_element_type=jnp.float32)
    h3 = lax.dot_general(xt, w3_v[slot].astype(jnp.bfloat16), dn,
                         preferred_element_type=jnp.float32)
    h = h1 * (1.0 / (1.0 + jnp.exp(-h1))) * h3
    o = lax.dot_general(h.astype(jnp.bfloat16), w2_v[slot].astype(jnp.bfloat16),
                        dn, preferred_element_type=jnp.float32)
    o = o * s_ref[...]

    tm1 = jnp.maximum(t - 1, 0)
    first = jnp.logical_or(t == 0, mt_ref[t] != mt_ref[tm1])

    @pl.when(first)
    def _():
        out_ref[...] = o

    @pl.when(jnp.logical_not(first))
    def _():
        out_ref[...] += o


def _grouped_ffn(routed_in, scores_sorted, w1, w2, w3, gid, mt, off_ext,
                 buf, flag, n_visits):
    nk, d = routed_in.shape
    e, h, _ = w1.shape

    grid_spec = pltpu.PrefetchScalarGridSpec(
        num_scalar_prefetch=5,
        grid=(n_visits,),
        in_specs=[
            pl.BlockSpec((_TM, d), lambda t, g, m, o, b, f: (m[t], 0)),
            pl.BlockSpec((_TM, 1), lambda t, g, m, o, b, f: (m[t], 0)),
            pl.BlockSpec(memory_space=pl.ANY),
            pl.BlockSpec(memory_space=pl.ANY),
            pl.BlockSpec(memory_space=pl.ANY),
        ],
        out_specs=pl.BlockSpec((_TM, d), lambda t, g, m, o, b, f: (m[t], 0)),
        scratch_shapes=[
            pltpu.VMEM((_NBUF, h, d), jnp.float32),
            pltpu.VMEM((_NBUF, h, d), jnp.float32),
            pltpu.VMEM((_NBUF, d, h), jnp.float32),
            pltpu.SemaphoreType.DMA((_NBUF, 3)),
        ],
    )
    return pl.pallas_call(
        _ffn_body,
        grid_spec=grid_spec,
        out_shape=jax.ShapeDtypeStruct((nk, d), jnp.float32),
        compiler_params=pltpu.CompilerParams(
            dimension_semantics=("arbitrary",)),
    )(gid, mt, off_ext, buf, flag, routed_in, scores_sorted, w1, w3, w2)


def kernel(x, top_scores, selected_experts_indices, w1, w2, w3):
    n, d = x.shape
    k = selected_experts_indices.shape[1]
    e = w1.shape[0]
    nk = n * k
    m_tiles = nk // _TM
    n_visits = m_tiles + e

    flat_e = selected_experts_indices.reshape(-1).astype(jnp.int32)
    sort_idx = jnp.argsort(flat_e, stable=True).astype(jnp.int32)
    tok = sort_idx // k
    inv = jnp.zeros((nk,), jnp.int32).at[sort_idx].set(
        jnp.arange(nk, dtype=jnp.int32))
    counts = jnp.bincount(flat_e, length=e).astype(jnp.int32)
    offsets = jnp.concatenate(
        [jnp.zeros((1,), jnp.int32), jnp.cumsum(counts).astype(jnp.int32)])
    off_ext = jnp.concatenate([offsets, jnp.full((1,), nk, jnp.int32)])

    ft = offsets[:e] // _TM
    lt = jnp.maximum(offsets[1:] - 1, 0) // _TM
    ntiles = jnp.where(counts > 0, lt - ft + 1, 0)
    vb = jnp.concatenate(
        [jnp.zeros((1,), jnp.int32), jnp.cumsum(ntiles).astype(jnp.int32)[:-1]])
    total = vb[-1] + ntiles[-1]
    marks = jnp.zeros((n_visits + 1,), jnp.int32).at[vb].add(1)
    gid = jnp.cumsum(marks)[:n_visits] - 1
    p = jnp.arange(n_visits, dtype=jnp.int32)
    valid = p < total
    gid_c = jnp.clip(gid, 0, e - 1)
    mt = ft[gid_c] + p - vb[gid_c]
    gid = jnp.where(valid, gid, e).astype(jnp.int32)
    mt = jnp.where(valid, jnp.clip(mt, 0, m_tiles - 1),
                   m_tiles - 1).astype(jnp.int32)

    gid_cc = jnp.minimum(gid, e - 1)
    flag = jnp.concatenate(
        [jnp.ones((1,), jnp.int32),
         (gid_cc[1:] != gid_cc[:-1]).astype(jnp.int32)])
    buf = ((jnp.cumsum(flag) - 1) % _NBUF).astype(jnp.int32)

    routed_in = _dispatch_gather(x, tok)

    scores_sorted = top_scores.reshape(-1)[sort_idx][:, None]
    routed_out = _grouped_ffn(routed_in, scores_sorted, w1, w2, w3, gid, mt,
                              off_ext, buf, flag, n_visits)

    return _combine(routed_out, inv)


# speedup vs baseline: 1.2228x; 1.0425x over previous
import functools

import jax
import jax.numpy as jnp
from jax import lax
from jax.experimental import pallas as pl
from jax.experimental.pallas import tpu as pltpu
from jax.experimental.pallas import tpu_sc as plsc

_TM = 256


def _dispatch_gather(x, tok):
    n, d = x.shape
    nk = tok.shape[0]
    info = plsc.get_sparse_core_info()
    nw = info.num_cores * info.num_subcores
    rows_w = nk // nw
    mesh = plsc.VectorSubcoreMesh(core_axis_name="c", subcore_axis_name="s")

    @functools.partial(
        pl.kernel, mesh=mesh,
        out_type=jax.ShapeDtypeStruct((nk, d), jnp.float32),
        scratch_types=[
            pltpu.VMEM((rows_w,), jnp.int32),
            pltpu.VMEM((rows_w, d), jnp.float32),
            pltpu.SemaphoreType.DMA,
        ])
    def k(x_hbm, tok_hbm, out_hbm, idx_v, rows_v, sem):
        wid = lax.axis_index("s") * info.num_cores + lax.axis_index("c")
        base = wid * rows_w
        pltpu.sync_copy(tok_hbm.at[pl.ds(base, rows_w)], idx_v)
        pltpu.async_copy(x_hbm.at[idx_v], rows_v, sem).wait()
        pltpu.sync_copy(rows_v, out_hbm.at[pl.ds(base, rows_w)])

    return k(x, tok)


def _combine(routed_out, inv):
    nk, d = routed_out.shape
    n = inv.shape[0] // 2
    info = plsc.get_sparse_core_info()
    nw = info.num_cores * info.num_subcores
    tok_w = n // nw
    tok_c = tok_w // 2
    mesh = plsc.VectorSubcoreMesh(core_axis_name="c", subcore_axis_name="s")

    @functools.partial(
        pl.kernel, mesh=mesh,
        out_type=jax.ShapeDtypeStruct((n, d), jnp.float32),
        scratch_types=[
            pltpu.VMEM((2 * tok_w,), jnp.int32),
            pltpu.VMEM((2 * tok_c, d), jnp.float32),
            pltpu.VMEM((2 * tok_c, d), jnp.float32),
            pltpu.VMEM((tok_c, d), jnp.float32),
            pltpu.SemaphoreType.DMA,
            pltpu.SemaphoreType.DMA,
            pltpu.SemaphoreType.DMA,
        ])
    def k(ro_hbm, inv_hbm, out_hbm, idx_v, rows0_v, rows1_v, out_v,
          gsem, osem0, osem1):
        wid = lax.axis_index("s") * info.num_cores + lax.axis_index("c")
        tbase = wid * tok_w
        pltpu.sync_copy(inv_hbm.at[pl.ds(2 * tbase, 2 * tok_w)], idx_v)
        g0 = pltpu.async_copy(ro_hbm.at[idx_v.at[pl.ds(0, 2 * tok_c)]],
                              rows0_v, gsem)
        g1 = pltpu.async_copy(ro_hbm.at[idx_v.at[pl.ds(2 * tok_c, 2 * tok_c)]],
                              rows1_v, gsem)
        g0.wait()

        @pl.loop(0, tok_c)
        def _(t):
            for j in range(0, d, 16):
                sl = pl.ds(j, 16)
                out_v[t, sl] = rows0_v[2 * t, sl] + rows0_v[2 * t + 1, sl]

        c0 = pltpu.async_copy(out_v, out_hbm.at[pl.ds(tbase, tok_c)], osem0)
        g1.wait()

        @pl.loop(0, tok_c)
        def _(t):
            for j in range(0, d, 16):
                sl = pl.ds(j, 16)
                rows1_v[2 * t, sl] = (rows1_v[2 * t, sl]
                                      + rows1_v[2 * t + 1, sl])

        c0.wait()

        @pl.loop(0, tok_c)
        def _(t):
            for j in range(0, d, 16):
                sl = pl.ds(j, 16)
                out_v[t, sl] = rows1_v[2 * t, sl]

        pltpu.async_copy(out_v, out_hbm.at[pl.ds(tbase + tok_c, tok_c)],
                         osem1).wait()

    return k(routed_out, inv)


_NBUF = 4


def _ffn_body(gid_ref, mt_ref, off_ref, buf_ref, flag_ref, x_ref, s_ref,
              w1_hbm, w3_hbm, w2_hbm, out_ref, w1_v, w3_v, w2_v, sems):
    t = pl.program_id(0)
    nv = pl.num_programs(0)
    ne = w1_hbm.shape[0]

    def issue(v):
        slot = buf_ref[v]
        ev = jnp.minimum(gid_ref[v], ne - 1)
        pltpu.make_async_copy(w1_hbm.at[ev], w1_v.at[slot],
                              sems.at[slot, 0]).start()
        pltpu.make_async_copy(w3_hbm.at[ev], w3_v.at[slot],
                              sems.at[slot, 1]).start()
        pltpu.make_async_copy(w2_hbm.at[ev], w2_v.at[slot],
                              sems.at[slot, 2]).start()

    la = _NBUF - 1

    @pl.when(t == 0)
    def _():
        issue(0)
        for v in range(1, la):
            @pl.when(jnp.logical_and(v < nv, flag_ref[v] == 1))
            def _(v=v):
                issue(v)

    @pl.when(jnp.logical_and(t + la < nv, flag_ref[t + la] == 1))
    def _():
        issue(t + la)

    @pl.when(flag_ref[t] == 1)
    def _():
        slot = buf_ref[t]
        ev = jnp.minimum(gid_ref[t], ne - 1)
        pltpu.make_async_copy(w1_hbm.at[ev], w1_v.at[slot],
                              sems.at[slot, 0]).wait()
        pltpu.make_async_copy(w3_hbm.at[ev], w3_v.at[slot],
                              sems.at[slot, 1]).wait()
        pltpu.make_async_copy(w2_hbm.at[ev], w2_v.at[slot],
                              sems.at[slot, 2]).wait()

    e = gid_ref[t]
    m = mt_ref[t]
    slot = buf_ref[t]
    start = off_ref[e]
    end = off_ref[e + 1]
    rows = m * _TM + lax.broadcasted_iota(jnp.int32, (_TM, 1), 0)
    mask = (rows >= start) & (rows < end)
    xt = jnp.where(mask, x_ref[...], 0.0).astype(jnp.bfloat16)
    dn = (((1,), (1,)), ((), ()))
    h1 = lax.dot_general(xt, w1_v[slot].astype(jnp.bfloat16), dn,
                         preferred_element_type=jnp.float32)
    h3 = lax.dot_general(xt, w3_v[slot].astype(jnp.bfloat16), dn,
                         preferred_element_type=jnp.float32)
    h = h1 * (1.0 / (1.0 + jnp.exp(-h1))) * h3
    o = lax.dot_general(h.astype(jnp.bfloat16), w2_v[slot].astype(jnp.bfloat16),
                        dn, preferred_element_type=jnp.float32)
    o = o * s_ref[...]

    tm1 = jnp.maximum(t - 1, 0)
    first = jnp.logical_or(t == 0, mt_ref[t] != mt_ref[tm1])

    @pl.when(first)
    def _():
        out_ref[...] = o

    @pl.when(jnp.logical_not(first))
    def _():
        out_ref[...] += o


def _grouped_ffn(routed_in, scores_sorted, w1, w2, w3, gid, mt, off_ext,
                 buf, flag, n_visits):
    nk, d = routed_in.shape
    e, h, _ = w1.shape

    grid_spec = pltpu.PrefetchScalarGridSpec(
        num_scalar_prefetch=5,
        grid=(n_visits,),
        in_specs=[
            pl.BlockSpec((_TM, d), lambda t, g, m, o, b, f: (m[t], 0)),
            pl.BlockSpec((_TM, 1), lambda t, g, m, o, b, f: (m[t], 0)),
            pl.BlockSpec(memory_space=pl.ANY),
            pl.BlockSpec(memory_space=pl.ANY),
            pl.BlockSpec(memory_space=pl.ANY),
        ],
        out_specs=pl.BlockSpec((_TM, d), lambda t, g, m, o, b, f: (m[t], 0)),
        scratch_shapes=[
            pltpu.VMEM((_NBUF, h, d), jnp.float32),
            pltpu.VMEM((_NBUF, h, d), jnp.float32),
            pltpu.VMEM((_NBUF, d, h), jnp.float32),
            pltpu.SemaphoreType.DMA((_NBUF, 3)),
        ],
    )
    return pl.pallas_call(
        _ffn_body,
        grid_spec=grid_spec,
        out_shape=jax.ShapeDtypeStruct((nk, d), jnp.float32),
        compiler_params=pltpu.CompilerParams(
            dimension_semantics=("arbitrary",)),
    )(gid, mt, off_ext, buf, flag, routed_in, scores_sorted, w1, w3, w2)


def kernel(x, top_scores, selected_experts_indices, w1, w2, w3):
    n, d = x.shape
    k = selected_experts_indices.shape[1]
    e = w1.shape[0]
    nk = n * k
    m_tiles = nk // _TM
    n_visits = m_tiles + e

    flat_e = selected_experts_indices.reshape(-1).astype(jnp.int32)
    sort_idx = jnp.argsort(flat_e, stable=True).astype(jnp.int32)
    tok = sort_idx // k
    inv = jnp.zeros((nk,), jnp.int32).at[sort_idx].set(
        jnp.arange(nk, dtype=jnp.int32))
    counts = jnp.bincount(flat_e, length=e).astype(jnp.int32)
    offsets = jnp.concatenate(
        [jnp.zeros((1,), jnp.int32), jnp.cumsum(counts).astype(jnp.int32)])
    off_ext = jnp.concatenate([offsets, jnp.full((1,), nk, jnp.int32)])

    ft = offsets[:e] // _TM
    lt = jnp.maximum(offsets[1:] - 1, 0) // _TM
    ntiles = jnp.where(counts > 0, lt - ft + 1, 0)
    vb = jnp.concatenate(
        [jnp.zeros((1,), jnp.int32), jnp.cumsum(ntiles).astype(jnp.int32)[:-1]])
    total = vb[-1] + ntiles[-1]
    marks = jnp.zeros((n_visits + 1,), jnp.int32).at[vb].add(1)
    gid = jnp.cumsum(marks)[:n_visits] - 1
    p = jnp.arange(n_visits, dtype=jnp.int32)
    valid = p < total
    gid_c = jnp.clip(gid, 0, e - 1)
    mt = ft[gid_c] + p - vb[gid_c]
    gid = jnp.where(valid, gid, e).astype(jnp.int32)
    mt = jnp.where(valid, jnp.clip(mt, 0, m_tiles - 1),
                   m_tiles - 1).astype(jnp.int32)

    gid_cc = jnp.minimum(gid, e - 1)
    flag = jnp.concatenate(
        [jnp.ones((1,), jnp.int32),
         (gid_cc[1:] != gid_cc[:-1]).astype(jnp.int32)])
    buf = ((jnp.cumsum(flag) - 1) % _NBUF).astype(jnp.int32)

    routed_in = _dispatch_gather(x, tok)

    scores_sorted = top_scores.reshape(-1)[sort_idx][:, None]
    routed_out = _grouped_ffn(routed_in, scores_sorted, w1, w2, w3, gid, mt,
                              off_ext, buf, flag, n_visits)

    return _combine(routed_out, inv)
